# baseline (speedup 1.0000x reference)
import functools
import math

import jax
import jax.numpy as jnp
from jax import lax
from jax.experimental import pallas as pl
from jax.experimental.pallas import tpu as pltpu
from jax.experimental.pallas import tpu_sc as plsc

D_MODEL = 2048
HEAD_DIM = 128
ROPE_THETA = 10000.0

NC, NS = 2, 16
NW = NC * NS
GROUP = 64


def _gather_body(rows_per_worker, idx_hbm, tbl_hbm, out_hbm,
                 idx_v, sem):
    wid = lax.axis_index("s") * NC + lax.axis_index("c")
    base = wid * rows_per_worker
    ngroups = rows_per_worker // GROUP
    pltpu.sync_copy(idx_hbm.at[wid], idx_v)

    def drain_one_group():
        pltpu.make_async_copy(tbl_hbm.at[pl.ds(0, GROUP)],
                              out_hbm.at[pl.ds(base, GROUP)], sem).wait()

    @pl.loop(0, ngroups)
    def _(g):
        for v16 in range(GROUP // 16):
            ids = idx_v[pl.ds(g * GROUP + v16 * 16, 16)]
            for r in range(16):
                pltpu.async_copy(
                    tbl_hbm.at[pl.ds(ids[r], 1)],
                    out_hbm.at[pl.ds(base + g * GROUP + v16 * 16 + r, 1)],
                    sem)

        @pl.when(g >= 1)
        def _():
            drain_one_group()

    drain_one_group()


def _emb_gather(ids_flat, weight):
    n_tok = ids_flat.shape[0]
    rows_per_worker = n_tok // NW
    ids2 = ids_flat.reshape(NW, rows_per_worker)
    mesh = plsc.VectorSubcoreMesh(core_axis_name="c", subcore_axis_name="s")
    k = pl.kernel(
        functools.partial(_gather_body, rows_per_worker),
        out_type=jax.ShapeDtypeStruct((n_tok, D_MODEL), jnp.float32),
        mesh=mesh,
        scratch_types=[
            pltpu.VMEM((rows_per_worker,), jnp.int32),
            pltpu.SemaphoreType.DMA,
        ],
    )
    return k(ids2, weight)


def _mask_body(block_rows, seqlen, o_ref):
    i = pl.program_id(0)
    r = lax.broadcasted_iota(jnp.int32, (block_rows, seqlen), 0) + i * block_rows
    c = lax.broadcasted_iota(jnp.int32, (block_rows, seqlen), 1)
    o_ref[...] = jnp.where(c > r, float("-inf"), 0.0).astype(jnp.float32)


def _causal_mask(seqlen):
    block_rows = 256
    return pl.pallas_call(
        functools.partial(_mask_body, block_rows, seqlen),
        out_shape=jax.ShapeDtypeStruct((seqlen, seqlen), jnp.float32),
        grid=(seqlen // block_rows,),
        out_specs=pl.BlockSpec((block_rows, seqlen), lambda i: (i, 0)),
    )()


def _freqs_body(end, half, o_ref):
    t = lax.broadcasted_iota(jnp.int32, (end, 2 * half), 0).astype(jnp.float32)
    c = lax.broadcasted_iota(jnp.int32, (end, 2 * half), 1)
    k = jnp.where(c < half, c, c - half).astype(jnp.float32)
    inv = jnp.exp(k * (-2.0 * math.log(ROPE_THETA) / HEAD_DIM))
    shift = jnp.where(c < half, 0.0, 0.5 * math.pi).astype(jnp.float32)
    o_ref[...] = jnp.cos(t * inv - shift)


def _freqs_cis(end):
    half = HEAD_DIM // 2
    cs = pl.pallas_call(
        functools.partial(_freqs_body, end, half),
        out_shape=jax.ShapeDtypeStruct((end, 2 * half), jnp.float32),
    )()
    return jax.lax.complex(cs[:, :half], cs[:, half:])


def kernel(input_ids, labels, weight):
    bsz, seqlen = input_ids.shape
    flat = _emb_gather(input_ids.reshape(bsz * seqlen), weight)
    hidden = flat.reshape(bsz, seqlen, D_MODEL)
    mask = _causal_mask(seqlen)
    freqs = _freqs_cis(4096)
    return (hidden, freqs, mask, labels)


# speedup vs baseline: 29.5331x; 29.5331x over previous
import functools
import math

import jax
import jax.numpy as jnp
from jax import lax
from jax.experimental import pallas as pl
from jax.experimental.pallas import tpu as pltpu
from jax.experimental.pallas import tpu_sc as plsc

D_MODEL = 2048
HEAD_DIM = 128
ROPE_THETA = 10000.0

NC, NS = 2, 16
NW = NC * NS
CHUNK = 8
NBUF = 4


def _gather_body(rows_per_worker, seqlen, idx_hbm, tbl_hbm, out_hbm,
                 idx_v, bufs, gs0, gs1, gs2, gs3, os0, os1, os2, os3):
    wid = lax.axis_index("s") * NC + lax.axis_index("c")
    workers_per_row = seqlen // rows_per_worker
    b0 = wid // workers_per_row
    s0 = (wid % workers_per_row) * rows_per_worker
    pltpu.sync_copy(idx_hbm.at[b0, pl.ds(s0, rows_per_worker)], idx_v)
    gsems = (gs0, gs1, gs2, gs3)
    osems = (os0, os1, os2, os3)
    CH = rows_per_worker // CHUNK

    def start_gather(j, b):
        pltpu.async_copy(tbl_hbm.at[idx_v.at[pl.ds(j * CHUNK, CHUNK)]],
                         bufs.at[b], gsems[b])

    def wait_gather(j, b):
        pltpu.make_async_copy(tbl_hbm.at[idx_v.at[pl.ds(j * CHUNK, CHUNK)]],
                              bufs.at[b], gsems[b]).wait()

    def start_out(j, b):
        pltpu.async_copy(bufs.at[b],
                         out_hbm.at[b0, pl.ds(s0 + j * CHUNK, CHUNK)],
                         osems[b])

    def wait_out(j, b):
        pltpu.make_async_copy(bufs.at[b],
                              out_hbm.at[b0, pl.ds(s0 + j * CHUNK, CHUNK)],
                              osems[b]).wait()

    start_gather(0, 0)
    start_gather(1, 1)
    for j in (0, 1):
        wait_gather(j, j)
        start_out(j, j)
        start_gather(j + 2, j + 2)

    @pl.loop(2, CH - 2, step=NBUF)
    def _(g):
        for db in range(NBUF):
            j = g + db
            b_cur = (2 + db) % NBUF
            b_nxt = db % NBUF
            wait_gather(j, b_cur)
            start_out(j, b_cur)
            wait_out(j - 2, b_nxt)
            start_gather(j + 2, b_nxt)

    for j in (CH - 2, CH - 1):
        b_cur = j % NBUF
        wait_gather(j, b_cur)
        start_out(j, b_cur)
        wait_out(j - 2, (j - 2) % NBUF)
    for j in (CH - 2, CH - 1):
        wait_out(j, j % NBUF)


def _emb_gather(input_ids, weight):
    bsz, seqlen = input_ids.shape
    rows_per_worker = bsz * seqlen // NW
    mesh = plsc.VectorSubcoreMesh(core_axis_name="c", subcore_axis_name="s")
    k = pl.kernel(
        functools.partial(_gather_body, rows_per_worker, seqlen),
        out_type=jax.ShapeDtypeStruct((bsz, seqlen, D_MODEL), jnp.float32),
        mesh=mesh,
        scratch_types=[
            pltpu.VMEM((rows_per_worker,), jnp.int32),
            pltpu.VMEM((NBUF, CHUNK, D_MODEL), jnp.float32),
        ] + [pltpu.SemaphoreType.DMA] * (2 * NBUF),
    )
    return k(input_ids, weight)


def _mask_freqs_body(block_rows, seqlen, half, mask_ref, cs_ref):
    i = pl.program_id(0)
    r = lax.broadcasted_iota(jnp.int32, (block_rows, seqlen), 0) + i * block_rows
    c = lax.broadcasted_iota(jnp.int32, (block_rows, seqlen), 1)
    mask_ref[...] = jnp.where(c > r, float("-inf"), 0.0).astype(jnp.float32)

    t = (lax.broadcasted_iota(jnp.int32, (block_rows, 2 * half), 0)
         + i * block_rows).astype(jnp.float32)
    fc = lax.broadcasted_iota(jnp.int32, (block_rows, 2 * half), 1)
    k = jnp.where(fc < half, fc, fc - half).astype(jnp.float32)
    inv = jnp.exp(k * (-2.0 * math.log(ROPE_THETA) / HEAD_DIM))
    shift = jnp.where(fc < half, 0.0, 0.5 * math.pi).astype(jnp.float32)
    cs_ref[...] = jnp.cos(t * inv - shift)


def _mask_and_freqs(seqlen):
    block_rows = 256
    half = HEAD_DIM // 2
    mask, cs = pl.pallas_call(
        functools.partial(_mask_freqs_body, block_rows, seqlen, half),
        out_shape=[
            jax.ShapeDtypeStruct((seqlen, seqlen), jnp.float32),
            jax.ShapeDtypeStruct((seqlen, 2 * half), jnp.float32),
        ],
        grid=(seqlen // block_rows,),
        out_specs=[
            pl.BlockSpec((block_rows, seqlen), lambda i: (i, 0)),
            pl.BlockSpec((block_rows, 2 * half), lambda i: (i, 0)),
        ],
    )()
    return mask, jax.lax.complex(cs[:, :half], cs[:, half:])


def kernel(input_ids, labels, weight):
    bsz, seqlen = input_ids.shape
    hidden = _emb_gather(input_ids, weight)
    mask, freqs = _mask_and_freqs(seqlen)
    return (hidden, freqs, mask, labels)


# speedup vs baseline: 29.7420x; 1.0071x over previous
import functools
import math

import jax
import jax.numpy as jnp
from jax import lax
from jax.experimental import pallas as pl
from jax.experimental.pallas import tpu as pltpu
from jax.experimental.pallas import tpu_sc as plsc

D_MODEL = 2048
HEAD_DIM = 128
ROPE_THETA = 10000.0

NC, NS = 2, 16
NW = NC * NS
CHUNK = 8
L_G = 3
L_W = 3
NBUF = L_G + L_W


def _gather_body(rows_per_worker, seqlen, idx_hbm, tbl_hbm, out_hbm,
                 idx_v, bufs, *sems):
    wid = lax.axis_index("s") * NC + lax.axis_index("c")
    workers_per_row = seqlen // rows_per_worker
    b0 = wid // workers_per_row
    s0 = (wid % workers_per_row) * rows_per_worker
    pltpu.sync_copy(idx_hbm.at[b0, pl.ds(s0, rows_per_worker)], idx_v)
    gsems = sems[:NBUF]
    osems = sems[NBUF:]
    CH = rows_per_worker // CHUNK

    def start_gather(j, b):
        pltpu.async_copy(tbl_hbm.at[idx_v.at[pl.ds(j * CHUNK, CHUNK)]],
                         bufs.at[b], gsems[b])

    def wait_gather(j, b):
        pltpu.make_async_copy(tbl_hbm.at[idx_v.at[pl.ds(j * CHUNK, CHUNK)]],
                              bufs.at[b], gsems[b]).wait()

    def start_out(j, b):
        pltpu.async_copy(bufs.at[b],
                         out_hbm.at[b0, pl.ds(s0 + j * CHUNK, CHUNK)],
                         osems[b])

    def wait_out(j, b):
        pltpu.make_async_copy(bufs.at[b],
                              out_hbm.at[b0, pl.ds(s0 + j * CHUNK, CHUNK)],
                              osems[b]).wait()

    K = (CH - L_W - L_G) // NBUF

    for j in range(L_G):
        start_gather(j, j)
    for j in range(L_W):
        wait_gather(j, j)
        start_out(j, j)
        start_gather(j + L_G, (j + L_G) % NBUF)

    @pl.loop(L_W, L_W + K * NBUF, step=NBUF)
    def _(g):
        for db in range(NBUF):
            j = g + db
            b = (L_W + db) % NBUF
            wait_gather(j, b)
            start_out(j, b)
            wait_out(j - L_W, (b - L_W) % NBUF)
            start_gather(j + L_G, (b + L_G) % NBUF)

    for j in range(L_W + K * NBUF, CH):
        b = j % NBUF
        wait_gather(j, b)
        start_out(j, b)
        wait_out(j - L_W, (j - L_W) % NBUF)
        if j + L_G < CH:
            start_gather(j + L_G, (j + L_G) % NBUF)
    for j in range(CH - L_W, CH):
        wait_out(j, j % NBUF)


def _emb_gather(input_ids, weight):
    bsz, seqlen = input_ids.shape
    rows_per_worker = bsz * seqlen // NW
    mesh = plsc.VectorSubcoreMesh(core_axis_name="c", subcore_axis_name="s")
    k = pl.kernel(
        functools.partial(_gather_body, rows_per_worker, seqlen),
        out_type=jax.ShapeDtypeStruct((bsz, seqlen, D_MODEL), jnp.float32),
        mesh=mesh,
        scratch_types=[
            pltpu.VMEM((rows_per_worker,), jnp.int32),
            pltpu.VMEM((NBUF, CHUNK, D_MODEL), jnp.float32),
        ] + [pltpu.SemaphoreType.DMA] * (2 * NBUF),
    )
    return k(input_ids, weight)


def _mask_freqs_body(block_rows, seqlen, half, mask_ref, cs_ref):
    i = pl.program_id(0)
    r = lax.broadcasted_iota(jnp.int32, (block_rows, seqlen), 0) + i * block_rows
    c = lax.broadcasted_iota(jnp.int32, (block_rows, seqlen), 1)
    mask_ref[...] = jnp.where(c > r, float("-inf"), 0.0).astype(jnp.float32)

    t = (lax.broadcasted_iota(jnp.int32, (block_rows, 2 * half), 0)
         + i * block_rows).astype(jnp.float32)
    fc = lax.broadcasted_iota(jnp.int32, (block_rows, 2 * half), 1)
    k = jnp.where(fc < half, fc, fc - half).astype(jnp.float32)
    inv = jnp.exp(k * (-2.0 * math.log(ROPE_THETA) / HEAD_DIM))
    shift = jnp.where(fc < half, 0.0, 0.5 * math.pi).astype(jnp.float32)
    cs_ref[...] = jnp.cos(t * inv - shift)


def _mask_and_freqs(seqlen):
    block_rows = 256
    half = HEAD_DIM // 2
    mask, cs = pl.pallas_call(
        functools.partial(_mask_freqs_body, block_rows, seqlen, half),
        out_shape=[
            jax.ShapeDtypeStruct((seqlen, seqlen), jnp.float32),
            jax.ShapeDtypeStruct((seqlen, 2 * half), jnp.float32),
        ],
        grid=(seqlen // block_rows,),
        out_specs=[
            pl.BlockSpec((block_rows, seqlen), lambda i: (i, 0)),
            pl.BlockSpec((block_rows, 2 * half), lambda i: (i, 0)),
        ],
    )()
    return mask, jax.lax.complex(cs[:, :half], cs[:, half:])


def kernel(input_ids, labels, weight):
    bsz, seqlen = input_ids.shape
    hidden = _emb_gather(input_ids, weight)
    mask, freqs = _mask_and_freqs(seqlen)
    return (hidden, freqs, mask, labels)


# speedup vs baseline: 29.8121x; 1.0024x over previous
import functools
import math

import jax
import jax.numpy as jnp
from jax import lax
from jax.experimental import pallas as pl
from jax.experimental.pallas import tpu as pltpu
from jax.experimental.pallas import tpu_sc as plsc

D_MODEL = 2048
HEAD_DIM = 128
ROPE_THETA = 10000.0

NC, NS = 2, 16
NW = NC * NS
CHUNK = 8
L_G = 2
L_W = 4
NBUF = L_G + L_W


def _gather_body(rows_per_worker, seqlen, idx_hbm, tbl_hbm, out_hbm,
                 idx_v, bufs, *sems):
    wid = lax.axis_index("s") * NC + lax.axis_index("c")
    workers_per_row = seqlen // rows_per_worker
    b0 = wid // workers_per_row
    s0 = (wid % workers_per_row) * rows_per_worker
    pltpu.sync_copy(idx_hbm.at[b0, pl.ds(s0, rows_per_worker)], idx_v)
    gsems = sems[:NBUF]
    osems = sems[NBUF:]
    CH = rows_per_worker // CHUNK

    def start_gather(j, b):
        pltpu.async_copy(tbl_hbm.at[idx_v.at[pl.ds(j * CHUNK, CHUNK)]],
                         bufs.at[b], gsems[b])

    def wait_gather(j, b):
        pltpu.make_async_copy(tbl_hbm.at[idx_v.at[pl.ds(j * CHUNK, CHUNK)]],
                              bufs.at[b], gsems[b]).wait()

    def start_out(j, b):
        pltpu.async_copy(bufs.at[b],
                         out_hbm.at[b0, pl.ds(s0 + j * CHUNK, CHUNK)],
                         osems[b])

    def wait_out(j, b):
        pltpu.make_async_copy(bufs.at[b],
                              out_hbm.at[b0, pl.ds(s0 + j * CHUNK, CHUNK)],
                              osems[b]).wait()

    K = (CH - L_W - L_G) // NBUF

    for j in range(L_G):
        start_gather(j, j)
    for j in range(L_W):
        wait_gather(j, j)
        start_out(j, j)
        start_gather(j + L_G, (j + L_G) % NBUF)

    @pl.loop(L_W, L_W + K * NBUF, step=NBUF)
    def _(g):
        for db in range(NBUF):
            j = g + db
            b = (L_W + db) % NBUF
            wait_gather(j, b)
            start_out(j, b)
            wait_out(j - L_W, (b - L_W) % NBUF)
            start_gather(j + L_G, (b + L_G) % NBUF)

    for j in range(L_W + K * NBUF, CH):
        b = j % NBUF
        wait_gather(j, b)
        start_out(j, b)
        wait_out(j - L_W, (j - L_W) % NBUF)
        if j + L_G < CH:
            start_gather(j + L_G, (j + L_G) % NBUF)
    for j in range(CH - L_W, CH):
        wait_out(j, j % NBUF)


def _emb_gather(input_ids, weight):
    bsz, seqlen = input_ids.shape
    rows_per_worker = bsz * seqlen // NW
    mesh = plsc.VectorSubcoreMesh(core_axis_name="c", subcore_axis_name="s")
    k = pl.kernel(
        functools.partial(_gather_body, rows_per_worker, seqlen),
        out_type=jax.ShapeDtypeStruct((bsz, seqlen, D_MODEL), jnp.float32),
        mesh=mesh,
        scratch_types=[
            pltpu.VMEM((rows_per_worker,), jnp.int32),
            pltpu.VMEM((NBUF, CHUNK, D_MODEL), jnp.float32),
        ] + [pltpu.SemaphoreType.DMA] * (2 * NBUF),
    )
    return k(input_ids, weight)


def _mask_freqs_body(block_rows, seqlen, half, mask_ref, cs_ref):
    i = pl.program_id(0)
    r = lax.broadcasted_iota(jnp.int32, (block_rows, seqlen), 0) + i * block_rows
    c = lax.broadcasted_iota(jnp.int32, (block_rows, seqlen), 1)
    mask_ref[...] = jnp.where(c > r, float("-inf"), 0.0).astype(jnp.float32)

    t = (lax.broadcasted_iota(jnp.int32, (block_rows, 2 * half), 0)
         + i * block_rows).astype(jnp.float32)
    fc = lax.broadcasted_iota(jnp.int32, (block_rows, 2 * half), 1)
    k = jnp.where(fc < half, fc, fc - half).astype(jnp.float32)
    inv = jnp.exp(k * (-2.0 * math.log(ROPE_THETA) / HEAD_DIM))
    shift = jnp.where(fc < half, 0.0, 0.5 * math.pi).astype(jnp.float32)
    cs_ref[...] = jnp.cos(t * inv - shift)


def _mask_and_freqs(seqlen):
    block_rows = 256
    half = HEAD_DIM // 2
    mask, cs = pl.pallas_call(
        functools.partial(_mask_freqs_body, block_rows, seqlen, half),
        out_shape=[
            jax.ShapeDtypeStruct((seqlen, seqlen), jnp.float32),
            jax.ShapeDtypeStruct((seqlen, 2 * half), jnp.float32),
        ],
        grid=(seqlen // block_rows,),
        out_specs=[
            pl.BlockSpec((block_rows, seqlen), lambda i: (i, 0)),
            pl.BlockSpec((block_rows, 2 * half), lambda i: (i, 0)),
        ],
    )()
    return mask, jax.lax.complex(cs[:, :half], cs[:, half:])


def kernel(input_ids, labels, weight):
    bsz, seqlen = input_ids.shape
    hidden = _emb_gather(input_ids, weight)
    mask, freqs = _mask_and_freqs(seqlen)
    return (hidden, freqs, mask, labels)
